# Initial kernel scaffold; baseline (speedup 1.0000x reference)
import jax
import jax.numpy as jnp
from jax.experimental import pallas as pl


def kernel(node_feature, edge_src, edge_dst):
    raise NotImplementedError("write your pallas kernel here")


# speedup vs baseline: 4.9258x; 4.9258x over previous
import functools

import jax
import jax.numpy as jnp
from jax import lax
from jax.experimental import pallas as pl
from jax.experimental.pallas import tpu as pltpu
from jax.experimental.pallas import tpu_sc as plsc

N_NODES = 10000
N_EDGES = 320000
D_FEAT = 128

_CHUNK = 128
_NCHUNK = N_EDGES // _CHUNK
_NW = 32


def _gather_incident_kernel(table_hbm, esrc_hbm, edst_hbm, out_hbm,
                            idx_d, idx_s, rows_d, rows_s, sem0, sem1):
    wid = lax.axis_index("s") * 2 + lax.axis_index("c")
    n_g = _NCHUNK // _NW + jnp.where(wid < _NCHUNK % _NW, 1, 0)

    def body(g, carry):
        chunk = g * _NW + wid
        base = chunk * _CHUNK
        pltpu.sync_copy(edst_hbm.at[pl.ds(base, _CHUNK)], idx_d)
        pltpu.sync_copy(esrc_hbm.at[pl.ds(base, _CHUNK)], idx_s)
        cp_d = pltpu.async_copy(table_hbm.at[idx_d], rows_d, sem0)
        cp_s = pltpu.async_copy(table_hbm.at[idx_s], rows_s, sem1)
        cp_d.wait()
        cp_s.wait()
        pltpu.sync_copy(rows_d, out_hbm.at[pl.ds(base, _CHUNK), pl.ds(0, D_FEAT)])
        pltpu.sync_copy(rows_s, out_hbm.at[pl.ds(base, _CHUNK), pl.ds(D_FEAT, D_FEAT)])
        return carry

    lax.fori_loop(0, n_g, body, 0)


@jax.jit
def kernel(node_feature, edge_src, edge_dst):
    mesh = plsc.VectorSubcoreMesh(core_axis_name="c", subcore_axis_name="s")
    run = pl.kernel(
        _gather_incident_kernel,
        out_type=jax.ShapeDtypeStruct((N_EDGES, 2 * D_FEAT), jnp.float32),
        mesh=mesh,
        scratch_types=[
            pltpu.VMEM((_CHUNK,), jnp.int32),
            pltpu.VMEM((_CHUNK,), jnp.int32),
            pltpu.VMEM((_CHUNK, D_FEAT), jnp.float32),
            pltpu.VMEM((_CHUNK, D_FEAT), jnp.float32),
            pltpu.SemaphoreType.DMA,
            pltpu.SemaphoreType.DMA,
        ],
    )
    return run(node_feature, edge_src, edge_dst)


# speedup vs baseline: 7.3550x; 1.4932x over previous
import jax
import jax.numpy as jnp
from jax import lax
from jax.experimental import pallas as pl
from jax.experimental.pallas import tpu as pltpu
from jax.experimental.pallas import tpu_sc as plsc

N_NODES = 10000
N_EDGES = 320000
D_FEAT = 128

_CHUNK = 128
_NCHUNK = N_EDGES // _CHUNK
_NW = 32
_NG_MAX = -(-_NCHUNK // _NW)


def _gather_incident_kernel(table_hbm, esrc_hbm, edst_hbm, out_hbm,
                            idx_d, idx_s, rows_d, rows_s,
                            semi_d, semi_s, semg_d, semg_s, semo_d, semo_s):
    wid = lax.axis_index("s") * 2 + lax.axis_index("c")
    n_g = _NCHUNK // _NW + jnp.where(wid < _NCHUNK % _NW, 1, 0)

    def chunk_base(g):
        return (g * _NW + wid) * _CHUNK

    def stage_a(g, b):
        @pl.when(jnp.logical_and(g >= 0, g < n_g))
        def _():
            base = chunk_base(g)
            pltpu.async_copy(edst_hbm.at[pl.ds(base, _CHUNK)], idx_d.at[b], semi_d[b])
            pltpu.async_copy(esrc_hbm.at[pl.ds(base, _CHUNK)], idx_s.at[b], semi_s[b])

    def stage_b(g, b):
        @pl.when(jnp.logical_and(g >= 0, g < n_g))
        def _():
            pltpu.make_async_copy(edst_hbm.at[pl.ds(0, _CHUNK)], idx_d.at[b], semi_d[b]).wait()
            pltpu.make_async_copy(esrc_hbm.at[pl.ds(0, _CHUNK)], idx_s.at[b], semi_s[b]).wait()

            @pl.when(g >= 2)
            def _():
                pltpu.make_async_copy(rows_d.at[b], out_hbm.at[pl.ds(0, _CHUNK), pl.ds(0, D_FEAT)], semo_d[b]).wait()
                pltpu.make_async_copy(rows_s.at[b], out_hbm.at[pl.ds(0, _CHUNK), pl.ds(D_FEAT, D_FEAT)], semo_s[b]).wait()

            pltpu.async_copy(table_hbm.at[idx_d.at[b]], rows_d.at[b], semg_d[b])
            pltpu.async_copy(table_hbm.at[idx_s.at[b]], rows_s.at[b], semg_s[b])

    def stage_c(g, b):
        @pl.when(jnp.logical_and(g >= 0, g < n_g))
        def _():
            pltpu.make_async_copy(table_hbm.at[idx_d.at[b]], rows_d.at[b], semg_d[b]).wait()
            pltpu.make_async_copy(table_hbm.at[idx_s.at[b]], rows_s.at[b], semg_s[b]).wait()
            base = chunk_base(g)
            pltpu.async_copy(rows_d.at[b], out_hbm.at[pl.ds(base, _CHUNK), pl.ds(0, D_FEAT)], semo_d[b])
            pltpu.async_copy(rows_s.at[b], out_hbm.at[pl.ds(base, _CHUNK), pl.ds(D_FEAT, D_FEAT)], semo_s[b])

    def step(s, carry):
        for p in range(2):
            g = s * 2 + p
            stage_b(g - 1, (p + 1) % 2)
            stage_c(g - 2, p % 2)
            stage_a(g, p % 2)
        return carry

    lax.fori_loop(0, (_NG_MAX + 2 + 1) // 2, step, 0)

    for b in range(2):
        @pl.when(n_g >= 2 - b)
        def _():
            pltpu.make_async_copy(rows_d.at[b], out_hbm.at[pl.ds(0, _CHUNK), pl.ds(0, D_FEAT)], semo_d[b]).wait()
            pltpu.make_async_copy(rows_s.at[b], out_hbm.at[pl.ds(0, _CHUNK), pl.ds(D_FEAT, D_FEAT)], semo_s[b]).wait()


@jax.jit
def kernel(node_feature, edge_src, edge_dst):
    mesh = plsc.VectorSubcoreMesh(core_axis_name="c", subcore_axis_name="s")
    run = pl.kernel(
        _gather_incident_kernel,
        out_type=jax.ShapeDtypeStruct((N_EDGES, 2 * D_FEAT), jnp.float32),
        mesh=mesh,
        scratch_types=[
            pltpu.VMEM((2, _CHUNK), jnp.int32),
            pltpu.VMEM((2, _CHUNK), jnp.int32),
            pltpu.VMEM((2, _CHUNK, D_FEAT), jnp.float32),
            pltpu.VMEM((2, _CHUNK, D_FEAT), jnp.float32),
            [pltpu.SemaphoreType.DMA] * 2,
            [pltpu.SemaphoreType.DMA] * 2,
            [pltpu.SemaphoreType.DMA] * 2,
            [pltpu.SemaphoreType.DMA] * 2,
            [pltpu.SemaphoreType.DMA] * 2,
            [pltpu.SemaphoreType.DMA] * 2,
        ],
    )
    return run(node_feature, edge_src, edge_dst)


# speedup vs baseline: 12.2231x; 1.6619x over previous
import jax
import jax.numpy as jnp
from jax import lax
from jax.experimental import pallas as pl
from jax.experimental.pallas import tpu as pltpu
from jax.experimental.pallas import tpu_sc as plsc

N_NODES = 10000
N_EDGES = 320000
D_FEAT = 128

_CHUNK = 80
_NCHUNK = N_EDGES // _CHUNK
_NW = 32
_NG_MAX = _NCHUNK // _NW


def _gather_incident_kernel(table_hbm, esrc_hbm, edst_hbm, out_hbm,
                            tbl_sh, idx_d, idx_s, rows_d, rows_s,
                            semi_d, semi_s, semg_d, semg_s, semo_d, semo_s):
    wid = lax.axis_index("s") * 2 + lax.axis_index("c")
    n_g = _NG_MAX

    @pl.when(lax.axis_index("s") == 0)
    def _():
        pltpu.sync_copy(table_hbm, tbl_sh)

    plsc.subcore_barrier()

    def chunk_base(g):
        return (g * _NW + wid) * _CHUNK

    def stage_a(g, b):
        @pl.when(jnp.logical_and(g >= 0, g < n_g))
        def _():
            base = chunk_base(g)
            pltpu.async_copy(edst_hbm.at[pl.ds(base, _CHUNK)], idx_d.at[b], semi_d[b])
            pltpu.async_copy(esrc_hbm.at[pl.ds(base, _CHUNK)], idx_s.at[b], semi_s[b])

    def stage_b(g, b):
        @pl.when(jnp.logical_and(g >= 0, g < n_g))
        def _():
            pltpu.make_async_copy(edst_hbm.at[pl.ds(0, _CHUNK)], idx_d.at[b], semi_d[b]).wait()
            pltpu.make_async_copy(esrc_hbm.at[pl.ds(0, _CHUNK)], idx_s.at[b], semi_s[b]).wait()

            @pl.when(g >= 2)
            def _():
                pltpu.make_async_copy(rows_d.at[b], out_hbm.at[pl.ds(0, _CHUNK), pl.ds(0, D_FEAT)], semo_d[b]).wait()
                pltpu.make_async_copy(rows_s.at[b], out_hbm.at[pl.ds(0, _CHUNK), pl.ds(D_FEAT, D_FEAT)], semo_s[b]).wait()

            pltpu.async_copy(tbl_sh.at[idx_d.at[b]], rows_d.at[b], semg_d[b])
            pltpu.async_copy(tbl_sh.at[idx_s.at[b]], rows_s.at[b], semg_s[b])

    def stage_c(g, b):
        @pl.when(jnp.logical_and(g >= 0, g < n_g))
        def _():
            pltpu.make_async_copy(tbl_sh.at[idx_d.at[b]], rows_d.at[b], semg_d[b]).wait()
            pltpu.make_async_copy(tbl_sh.at[idx_s.at[b]], rows_s.at[b], semg_s[b]).wait()
            base = chunk_base(g)
            pltpu.async_copy(rows_d.at[b], out_hbm.at[pl.ds(base, _CHUNK), pl.ds(0, D_FEAT)], semo_d[b])
            pltpu.async_copy(rows_s.at[b], out_hbm.at[pl.ds(base, _CHUNK), pl.ds(D_FEAT, D_FEAT)], semo_s[b])

    def step(s, carry):
        for p in range(2):
            g = s * 2 + p
            stage_b(g - 1, (p + 1) % 2)
            stage_c(g - 2, p % 2)
            stage_a(g, p % 2)
        return carry

    lax.fori_loop(0, (_NG_MAX + 2 + 1) // 2, step, 0)

    for b in range(2):
        @pl.when(n_g >= 2 - b)
        def _():
            pltpu.make_async_copy(rows_d.at[b], out_hbm.at[pl.ds(0, _CHUNK), pl.ds(0, D_FEAT)], semo_d[b]).wait()
            pltpu.make_async_copy(rows_s.at[b], out_hbm.at[pl.ds(0, _CHUNK), pl.ds(D_FEAT, D_FEAT)], semo_s[b]).wait()


@jax.jit
def kernel(node_feature, edge_src, edge_dst):
    mesh = plsc.VectorSubcoreMesh(core_axis_name="c", subcore_axis_name="s")
    run = pl.kernel(
        _gather_incident_kernel,
        out_type=jax.ShapeDtypeStruct((N_EDGES, 2 * D_FEAT), jnp.float32),
        mesh=mesh,
        scratch_types=[
            pltpu.VMEM_SHARED((N_NODES, D_FEAT), jnp.float32),
            pltpu.VMEM((2, _CHUNK), jnp.int32),
            pltpu.VMEM((2, _CHUNK), jnp.int32),
            pltpu.VMEM((2, _CHUNK, D_FEAT), jnp.float32),
            pltpu.VMEM((2, _CHUNK, D_FEAT), jnp.float32),
            [pltpu.SemaphoreType.DMA] * 2,
            [pltpu.SemaphoreType.DMA] * 2,
            [pltpu.SemaphoreType.DMA] * 2,
            [pltpu.SemaphoreType.DMA] * 2,
            [pltpu.SemaphoreType.DMA] * 2,
            [pltpu.SemaphoreType.DMA] * 2,
        ],
    )
    return run(node_feature, edge_src, edge_dst)
